# baseline (speedup 1.0000x reference)
import functools

import jax
import jax.numpy as jnp
import numpy as np
from jax import lax
from jax.experimental import pallas as pl
from jax.experimental.pallas import tpu as pltpu
from jax.experimental.pallas import tpu_sc as plsc

RES = 256
FOV = 16.0
CMF_A = 0.5
SIGMA = 4.0
B, C, H, W = 32, 3, 512, 512
N = RES * RES
HW = H * W

NUM_CORES = 2
NUM_SUBCORES = 16
NW = NUM_CORES * NUM_SUBCORES
PTS = N // NW
LANES = 16
VSTEPS = PTS // LANES
GLEN = C * PTS


def _grid_tables():
    r_max = FOV / 2.0
    rho_max = np.log((r_max + CMF_A) / CMF_A)
    lin = np.linspace(-rho_max, rho_max, RES, dtype=np.float32)
    u, v = np.meshgrid(lin, lin, indexing="ij")
    rho = np.sqrt(u ** 2 + v ** 2) + 1e-8
    r = CMF_A * (np.exp(rho) - 1.0)
    r = np.minimum(r, r_max)
    vx = u / rho * r
    vy = v / rho * r
    coords = np.stack([vx.ravel(), vy.ravel()], axis=-1).astype(np.float32) / r_max
    radius = r.ravel().astype(np.float32)
    ay = (coords[:, 0] * np.float32(H - 1)) / np.float32(2.0)
    ax = (coords[:, 1] * np.float32(W - 1)) / np.float32(2.0)
    decay = np.exp(-radius / np.float32(SIGMA)).astype(np.float32)
    return ay.astype(np.float32), ax.astype(np.float32), decay


_AY, _AX, _DECAY = _grid_tables()


def _sc_body(xflat, fixflat, ay_h, ax_h, dec_h, out,
             ay_v, ax_v, dec_v, fix_v,
             idx_a, idx_b, g_a, g_b,
             gsem_a, gsem_b, osem_a, osem_b):
    wid = lax.axis_index("s") * NUM_CORES + lax.axis_index("c")
    base = wid * PTS

    pltpu.sync_copy(ay_h.at[pl.ds(base, PTS)], ay_v)
    pltpu.sync_copy(ax_h.at[pl.ds(base, PTS)], ax_v)
    pltpu.sync_copy(dec_h.at[pl.ds(base, PTS)], dec_v)
    pltpu.sync_copy(fixflat, fix_v)

    def compute_idx(b, idx_v):
        cy = fix_v[pl.ds((2 * b) * LANES, LANES)]
        cx = fix_v[pl.ds((2 * b + 1) * LANES, LANES)]
        bo = b * (C * HW)

        def idx_body(i, _):
            s = i * LANES
            ayv = ay_v[pl.ds(s, LANES)]
            axv = ax_v[pl.ds(s, LANES)]
            uy = jnp.minimum(jnp.maximum(cy + ayv, jnp.float32(0.5)),
                             jnp.float32(H - 1) + jnp.float32(0.5))
            ux = jnp.minimum(jnp.maximum(cx + axv, jnp.float32(0.5)),
                             jnp.float32(W - 1) + jnp.float32(0.5))
            py = uy.astype(jnp.int32)
            px = ux.astype(jnp.int32)
            py = py - jnp.where(py.astype(jnp.float32) == uy, py & 1, 0)
            px = px - jnp.where(px.astype(jnp.float32) == ux, px & 1, 0)
            flat = (py * W + px) * C + bo
            idx_v[pl.ds(s, LANES)] = flat
            idx_v[pl.ds(s + PTS, LANES)] = flat + 1
            idx_v[pl.ds(s + 2 * PTS, LANES)] = flat + 2
            return 0

        lax.fori_loop(0, VSTEPS, idx_body, 0)

    def fire_gather(idx_v, g_v, gsem):
        pltpu.async_copy(xflat.at[idx_v], g_v, gsem)

    def wait_gather(g_v, gsem):
        pltpu.make_async_copy(xflat.at[pl.ds(0, GLEN)], g_v, gsem).wait()

    def decay_mul(g_v):
        def dec_body(i, _):
            s = i * LANES
            d = dec_v[pl.ds(s, LANES)]
            g_v[pl.ds(s, LANES)] = g_v[pl.ds(s, LANES)] * d
            g_v[pl.ds(s + PTS, LANES)] = g_v[pl.ds(s + PTS, LANES)] * d
            g_v[pl.ds(s + 2 * PTS, LANES)] = g_v[pl.ds(s + 2 * PTS, LANES)] * d
            return 0

        lax.fori_loop(0, VSTEPS, dec_body, 0)

    def fire_out(b, g_v, osem):
        obase = b * (C * N) + base
        pltpu.async_copy(g_v.at[pl.ds(0, PTS)], out.at[pl.ds(obase, PTS)], osem)
        pltpu.async_copy(g_v.at[pl.ds(PTS, PTS)],
                         out.at[pl.ds(obase + N, PTS)], osem)
        pltpu.async_copy(g_v.at[pl.ds(2 * PTS, PTS)],
                         out.at[pl.ds(obase + 2 * N, PTS)], osem)

    def wait_out(g_v, osem):
        pltpu.make_async_copy(g_v, out.at[pl.ds(0, GLEN)], osem).wait()

    compute_idx(0, idx_a)
    fire_gather(idx_a, g_a, gsem_a)

    def body(k, _):
        b1 = 2 * k + 1
        b2 = 2 * k + 2
        compute_idx(b1, idx_b)

        @pl.when(k >= 1)
        def _():
            wait_out(g_b, osem_b)

        fire_gather(idx_b, g_b, gsem_b)
        wait_gather(g_a, gsem_a)
        decay_mul(g_a)
        fire_out(b1 - 1, g_a, osem_a)

        compute_idx(b2, idx_a)
        wait_out(g_a, osem_a)
        fire_gather(idx_a, g_a, gsem_a)
        wait_gather(g_b, gsem_b)
        decay_mul(g_b)
        fire_out(b1, g_b, osem_b)
        return 0

    lax.fori_loop(0, (B - 2) // 2, body, 0)

    compute_idx(B - 1, idx_b)
    wait_out(g_b, osem_b)
    fire_gather(idx_b, g_b, gsem_b)
    wait_gather(g_a, gsem_a)
    decay_mul(g_a)
    fire_out(B - 2, g_a, osem_a)
    wait_gather(g_b, gsem_b)
    decay_mul(g_b)
    fire_out(B - 1, g_b, osem_b)
    wait_out(g_a, osem_a)
    wait_out(g_b, osem_b)


@jax.jit
def kernel(x, fix_loc):
    xflat = jnp.transpose(x, (0, 2, 3, 1)).reshape(-1)
    scale = jnp.array([H - 1, W - 1], dtype=jnp.float32)
    fixflat = jnp.repeat((fix_loc * scale + jnp.float32(0.5)).reshape(-1),
                         LANES)
    ay = jnp.asarray(_AY)
    ax = jnp.asarray(_AX)
    dec = jnp.asarray(_DECAY)

    mesh = plsc.VectorSubcoreMesh(core_axis_name="c", subcore_axis_name="s")
    f = functools.partial(
        pl.kernel,
        out_type=jax.ShapeDtypeStruct((B * C * N,), jnp.float32),
        mesh=mesh,
        scratch_types=[
            pltpu.VMEM((PTS,), jnp.float32),
            pltpu.VMEM((PTS,), jnp.float32),
            pltpu.VMEM((PTS,), jnp.float32),
            pltpu.VMEM((2 * B * LANES,), jnp.float32),
            pltpu.VMEM((GLEN,), jnp.int32),
            pltpu.VMEM((GLEN,), jnp.int32),
            pltpu.VMEM((GLEN,), jnp.float32),
            pltpu.VMEM((GLEN,), jnp.float32),
            pltpu.SemaphoreType.DMA,
            pltpu.SemaphoreType.DMA,
            pltpu.SemaphoreType.DMA,
            pltpu.SemaphoreType.DMA,
        ],
    )(_sc_body)
    return f(xflat, fixflat, ay, ax, dec).reshape(B, C, N)


# speedup vs baseline: 18.7679x; 18.7679x over previous
import functools

import jax
import jax.numpy as jnp
import numpy as np
from jax import lax
from jax.experimental import pallas as pl
from jax.experimental.pallas import tpu as pltpu
from jax.experimental.pallas import tpu_sc as plsc

RES = 256
FOV = 16.0
CMF_A = 0.5
SIGMA = 4.0
B, C, H, W = 32, 3, 512, 512
N = RES * RES
HW = H * W
CHW = C * HW

NUM_CORES = 2
NUM_SUBCORES = 16
NW = NUM_CORES * NUM_SUBCORES
PTS = N // NW
LANES = 16
VSTEPS = PTS // LANES
GLEN = C * PTS
SCHUNK = CHW // NUM_SUBCORES


def _grid_tables():
    r_max = FOV / 2.0
    rho_max = np.log((r_max + CMF_A) / CMF_A)
    lin = np.linspace(-rho_max, rho_max, RES, dtype=np.float32)
    u, v = np.meshgrid(lin, lin, indexing="ij")
    rho = np.sqrt(u ** 2 + v ** 2) + 1e-8
    r = CMF_A * (np.exp(rho) - 1.0)
    r = np.minimum(r, r_max)
    vx = u / rho * r
    vy = v / rho * r
    coords = np.stack([vx.ravel(), vy.ravel()], axis=-1).astype(np.float32) / r_max
    radius = r.ravel().astype(np.float32)
    ay = (coords[:, 0] * np.float32(H - 1)) / np.float32(2.0)
    ax = (coords[:, 1] * np.float32(W - 1)) / np.float32(2.0)
    decay = np.exp(-radius / np.float32(SIGMA)).astype(np.float32)
    return ay.astype(np.float32), ax.astype(np.float32), decay


_AY, _AX, _DECAY = _grid_tables()


def _sc_body(xflat, fixflat, ay_h, ax_h, dec_h, out,
             xs, ay_v, ax_v, dec_v, fix_v,
             idx_a, idx_b, g_a, g_b,
             gsem_a, gsem_b, osem_a, osem_b, ssem):
    sid = lax.axis_index("s")
    wid = sid * NUM_CORES + lax.axis_index("c")
    base = wid * PTS

    pltpu.sync_copy(ay_h.at[pl.ds(base, PTS)], ay_v)
    pltpu.sync_copy(ax_h.at[pl.ds(base, PTS)], ax_v)
    pltpu.sync_copy(dec_h.at[pl.ds(base, PTS)], dec_v)
    pltpu.sync_copy(fixflat, fix_v)

    def fire_stage(b):
        src = xflat.at[pl.ds(b * CHW + sid * SCHUNK, SCHUNK)]
        dst = xs.at[pl.ds((b & 1) * CHW + sid * SCHUNK, SCHUNK)]
        pltpu.async_copy(src, dst, ssem)

    def wait_stage():
        pltpu.make_async_copy(xflat.at[pl.ds(0, SCHUNK)],
                              xs.at[pl.ds(0, SCHUNK)], ssem).wait()

    def compute_idx(b, idx_v):
        cy = fix_v[pl.ds((2 * b) * LANES, LANES)]
        cx = fix_v[pl.ds((2 * b + 1) * LANES, LANES)]
        qo = (b & 1) * CHW

        def idx_body(i, _):
            s = i * LANES
            ayv = ay_v[pl.ds(s, LANES)]
            axv = ax_v[pl.ds(s, LANES)]
            uy = jnp.minimum(jnp.maximum(cy + ayv, jnp.float32(0.5)),
                             jnp.float32(H - 1) + jnp.float32(0.5))
            ux = jnp.minimum(jnp.maximum(cx + axv, jnp.float32(0.5)),
                             jnp.float32(W - 1) + jnp.float32(0.5))
            py = uy.astype(jnp.int32)
            px = ux.astype(jnp.int32)
            py = py - jnp.where(py.astype(jnp.float32) == uy, py & 1, 0)
            px = px - jnp.where(px.astype(jnp.float32) == ux, px & 1, 0)
            flat = py * W + px + qo
            idx_v[pl.ds(s, LANES)] = flat
            idx_v[pl.ds(s + PTS, LANES)] = flat + HW
            idx_v[pl.ds(s + 2 * PTS, LANES)] = flat + 2 * HW
            return 0

        lax.fori_loop(0, VSTEPS, idx_body, 0)

    def fire_gather(idx_v, g_v, gsem):
        pltpu.async_copy(xs.at[idx_v], g_v, gsem)

    def wait_gather(g_v, gsem):
        pltpu.make_async_copy(xflat.at[pl.ds(0, GLEN)], g_v, gsem).wait()

    def decay_mul(g_v):
        def dec_body(i, _):
            s = i * LANES
            d = dec_v[pl.ds(s, LANES)]
            g_v[pl.ds(s, LANES)] = g_v[pl.ds(s, LANES)] * d
            g_v[pl.ds(s + PTS, LANES)] = g_v[pl.ds(s + PTS, LANES)] * d
            g_v[pl.ds(s + 2 * PTS, LANES)] = g_v[pl.ds(s + 2 * PTS, LANES)] * d
            return 0

        lax.fori_loop(0, VSTEPS, dec_body, 0)

    def fire_out(b, g_v, osem):
        obase = b * (C * N) + base
        pltpu.async_copy(g_v.at[pl.ds(0, PTS)], out.at[pl.ds(obase, PTS)], osem)
        pltpu.async_copy(g_v.at[pl.ds(PTS, PTS)],
                         out.at[pl.ds(obase + N, PTS)], osem)
        pltpu.async_copy(g_v.at[pl.ds(2 * PTS, PTS)],
                         out.at[pl.ds(obase + 2 * N, PTS)], osem)

    def wait_out(g_v, osem):
        pltpu.make_async_copy(g_v, out.at[pl.ds(0, GLEN)], osem).wait()

    fire_stage(0)
    compute_idx(0, idx_a)
    wait_stage()
    plsc.subcore_barrier()
    fire_gather(idx_a, g_a, gsem_a)
    fire_stage(1)
    compute_idx(1, idx_b)
    wait_stage()
    plsc.subcore_barrier()
    fire_gather(idx_b, g_b, gsem_b)

    def body(k, _):
        r = 2 * k
        wait_gather(g_a, gsem_a)
        decay_mul(g_a)
        fire_out(r, g_a, osem_a)
        plsc.subcore_barrier()
        fire_stage(r + 2)
        compute_idx(r + 2, idx_a)
        wait_out(g_a, osem_a)
        wait_stage()
        plsc.subcore_barrier()
        fire_gather(idx_a, g_a, gsem_a)

        wait_gather(g_b, gsem_b)
        decay_mul(g_b)
        fire_out(r + 1, g_b, osem_b)
        plsc.subcore_barrier()
        fire_stage(r + 3)
        compute_idx(r + 3, idx_b)
        wait_out(g_b, osem_b)
        wait_stage()
        plsc.subcore_barrier()
        fire_gather(idx_b, g_b, gsem_b)
        return 0

    lax.fori_loop(0, (B - 2) // 2, body, 0)

    wait_gather(g_a, gsem_a)
    decay_mul(g_a)
    fire_out(B - 2, g_a, osem_a)
    wait_gather(g_b, gsem_b)
    decay_mul(g_b)
    fire_out(B - 1, g_b, osem_b)
    wait_out(g_a, osem_a)
    wait_out(g_b, osem_b)


@jax.jit
def kernel(x, fix_loc):
    xflat = x.reshape(-1)
    scale = jnp.array([H - 1, W - 1], dtype=jnp.float32)
    fixflat = jnp.repeat((fix_loc * scale + jnp.float32(0.5)).reshape(-1),
                         LANES)
    ay = jnp.asarray(_AY)
    ax = jnp.asarray(_AX)
    dec = jnp.asarray(_DECAY)

    mesh = plsc.VectorSubcoreMesh(core_axis_name="c", subcore_axis_name="s")
    f = functools.partial(
        pl.kernel,
        out_type=jax.ShapeDtypeStruct((B * C * N,), jnp.float32),
        mesh=mesh,
        scratch_types=[
            pltpu.VMEM_SHARED((2 * CHW,), jnp.float32),
            pltpu.VMEM((PTS,), jnp.float32),
            pltpu.VMEM((PTS,), jnp.float32),
            pltpu.VMEM((PTS,), jnp.float32),
            pltpu.VMEM((2 * B * LANES,), jnp.float32),
            pltpu.VMEM((GLEN,), jnp.int32),
            pltpu.VMEM((GLEN,), jnp.int32),
            pltpu.VMEM((GLEN,), jnp.float32),
            pltpu.VMEM((GLEN,), jnp.float32),
            pltpu.SemaphoreType.DMA,
            pltpu.SemaphoreType.DMA,
            pltpu.SemaphoreType.DMA,
            pltpu.SemaphoreType.DMA,
            pltpu.SemaphoreType.DMA,
        ],
    )(_sc_body)
    return f(xflat, fixflat, ay, ax, dec).reshape(B, C, N)


# speedup vs baseline: 20.5527x; 1.0951x over previous
import functools

import jax
import jax.numpy as jnp
import numpy as np
from jax import lax
from jax.experimental import pallas as pl
from jax.experimental.pallas import tpu as pltpu
from jax.experimental.pallas import tpu_sc as plsc

RES = 256
FOV = 16.0
CMF_A = 0.5
SIGMA = 4.0
B, C, H, W = 32, 3, 512, 512
N = RES * RES
HW = H * W
CHW = C * HW

NUM_CORES = 2
NUM_SUBCORES = 16
NW = NUM_CORES * NUM_SUBCORES
PTS = N // NW
LANES = 16
VSTEPS = PTS // LANES
GLEN = C * PTS
HW16 = HW // NUM_SUBCORES
SKEW = 128
CSTRIDE = HW + SKEW
SBUF = C * CSTRIDE


def _grid_tables():
    r_max = FOV / 2.0
    rho_max = np.log((r_max + CMF_A) / CMF_A)
    lin = np.linspace(-rho_max, rho_max, RES, dtype=np.float32)
    u, v = np.meshgrid(lin, lin, indexing="ij")
    rho = np.sqrt(u ** 2 + v ** 2) + 1e-8
    r = CMF_A * (np.exp(rho) - 1.0)
    r = np.minimum(r, r_max)
    vx = u / rho * r
    vy = v / rho * r
    coords = np.stack([vx.ravel(), vy.ravel()], axis=-1).astype(np.float32) / r_max
    radius = r.ravel().astype(np.float32)
    ay = (coords[:, 0] * np.float32(H - 1)) / np.float32(2.0)
    ax = (coords[:, 1] * np.float32(W - 1)) / np.float32(2.0)
    decay = np.exp(-radius / np.float32(SIGMA)).astype(np.float32)
    return ay.astype(np.float32), ax.astype(np.float32), decay


_AY, _AX, _DECAY = _grid_tables()


def _sc_body(xflat, fixflat, ay_h, ax_h, dec_h, out,
             xs, ay_v, ax_v, dec_v, fix_v,
             idx_a, idx_b, g_a, g_b,
             gsem_a, gsem_b, osem_a, osem_b, ssem):
    sid = lax.axis_index("s")
    wid = sid * NUM_CORES + lax.axis_index("c")
    base = wid * PTS

    pltpu.sync_copy(ay_h.at[pl.ds(base, PTS)], ay_v)
    pltpu.sync_copy(ax_h.at[pl.ds(base, PTS)], ax_v)
    pltpu.sync_copy(dec_h.at[pl.ds(base, PTS)], dec_v)
    pltpu.sync_copy(fixflat, fix_v)

    def fire_stage(b):
        for c in range(C):
            src = xflat.at[pl.ds(b * CHW + c * HW + sid * HW16, HW16)]
            dst = xs.at[pl.ds((b & 1) * SBUF + c * CSTRIDE + sid * HW16,
                              HW16)]
            pltpu.async_copy(src, dst, ssem)

    def wait_stage():
        pltpu.make_async_copy(xflat.at[pl.ds(0, C * HW16)],
                              xs.at[pl.ds(0, C * HW16)], ssem).wait()

    def compute_idx(b, idx_v):
        cy = fix_v[pl.ds((2 * b) * LANES, LANES)]
        cx = fix_v[pl.ds((2 * b + 1) * LANES, LANES)]
        qo = (b & 1) * SBUF

        def idx_body(i, _):
            s = i * LANES
            ayv = ay_v[pl.ds(s, LANES)]
            axv = ax_v[pl.ds(s, LANES)]
            uy = jnp.minimum(jnp.maximum(cy + ayv, jnp.float32(0.5)),
                             jnp.float32(H - 1) + jnp.float32(0.5))
            ux = jnp.minimum(jnp.maximum(cx + axv, jnp.float32(0.5)),
                             jnp.float32(W - 1) + jnp.float32(0.5))
            py = uy.astype(jnp.int32)
            px = ux.astype(jnp.int32)
            py = py - jnp.where(py.astype(jnp.float32) == uy, py & 1, 0)
            px = px - jnp.where(px.astype(jnp.float32) == ux, px & 1, 0)
            flat = py * W + px + qo
            idx_v[pl.ds(s, LANES)] = flat
            idx_v[pl.ds(s + PTS, LANES)] = flat + CSTRIDE
            idx_v[pl.ds(s + 2 * PTS, LANES)] = flat + 2 * CSTRIDE
            return 0

        lax.fori_loop(0, VSTEPS, idx_body, 0)

    def fire_gather(idx_v, g_v, gsem):
        pltpu.async_copy(xs.at[idx_v], g_v, gsem)

    def wait_gather(g_v, gsem):
        pltpu.make_async_copy(xflat.at[pl.ds(0, GLEN)], g_v, gsem).wait()

    def decay_mul(g_v):
        def dec_body(i, _):
            s = i * LANES
            d = dec_v[pl.ds(s, LANES)]
            g_v[pl.ds(s, LANES)] = g_v[pl.ds(s, LANES)] * d
            g_v[pl.ds(s + PTS, LANES)] = g_v[pl.ds(s + PTS, LANES)] * d
            g_v[pl.ds(s + 2 * PTS, LANES)] = g_v[pl.ds(s + 2 * PTS, LANES)] * d
            return 0

        lax.fori_loop(0, VSTEPS, dec_body, 0)

    def fire_out(b, g_v, osem):
        obase = b * (C * N) + base
        pltpu.async_copy(g_v.at[pl.ds(0, PTS)], out.at[pl.ds(obase, PTS)], osem)
        pltpu.async_copy(g_v.at[pl.ds(PTS, PTS)],
                         out.at[pl.ds(obase + N, PTS)], osem)
        pltpu.async_copy(g_v.at[pl.ds(2 * PTS, PTS)],
                         out.at[pl.ds(obase + 2 * N, PTS)], osem)

    def wait_out(g_v, osem):
        pltpu.make_async_copy(g_v, out.at[pl.ds(0, GLEN)], osem).wait()

    fire_stage(0)
    compute_idx(0, idx_a)
    wait_stage()
    plsc.subcore_barrier()
    fire_gather(idx_a, g_a, gsem_a)
    fire_stage(1)
    compute_idx(1, idx_b)
    wait_stage()
    plsc.subcore_barrier()
    fire_gather(idx_b, g_b, gsem_b)

    def body(k, _):
        r = 2 * k
        wait_gather(g_a, gsem_a)
        decay_mul(g_a)
        fire_out(r, g_a, osem_a)
        plsc.subcore_barrier()
        fire_stage(r + 2)
        compute_idx(r + 2, idx_a)
        wait_out(g_a, osem_a)
        wait_stage()
        plsc.subcore_barrier()
        fire_gather(idx_a, g_a, gsem_a)

        wait_gather(g_b, gsem_b)
        decay_mul(g_b)
        fire_out(r + 1, g_b, osem_b)
        plsc.subcore_barrier()
        fire_stage(r + 3)
        compute_idx(r + 3, idx_b)
        wait_out(g_b, osem_b)
        wait_stage()
        plsc.subcore_barrier()
        fire_gather(idx_b, g_b, gsem_b)
        return 0

    lax.fori_loop(0, (B - 2) // 2, body, 0)

    wait_gather(g_a, gsem_a)
    decay_mul(g_a)
    fire_out(B - 2, g_a, osem_a)
    wait_gather(g_b, gsem_b)
    decay_mul(g_b)
    fire_out(B - 1, g_b, osem_b)
    wait_out(g_a, osem_a)
    wait_out(g_b, osem_b)


@jax.jit
def kernel(x, fix_loc):
    xflat = x.reshape(-1)
    scale = jnp.array([H - 1, W - 1], dtype=jnp.float32)
    fixflat = jnp.repeat((fix_loc * scale + jnp.float32(0.5)).reshape(-1),
                         LANES)
    ay = jnp.asarray(_AY)
    ax = jnp.asarray(_AX)
    dec = jnp.asarray(_DECAY)

    mesh = plsc.VectorSubcoreMesh(core_axis_name="c", subcore_axis_name="s")
    f = functools.partial(
        pl.kernel,
        out_type=jax.ShapeDtypeStruct((B * C * N,), jnp.float32),
        mesh=mesh,
        scratch_types=[
            pltpu.VMEM_SHARED((2 * SBUF,), jnp.float32),
            pltpu.VMEM((PTS,), jnp.float32),
            pltpu.VMEM((PTS,), jnp.float32),
            pltpu.VMEM((PTS,), jnp.float32),
            pltpu.VMEM((2 * B * LANES,), jnp.float32),
            pltpu.VMEM((GLEN,), jnp.int32),
            pltpu.VMEM((GLEN,), jnp.int32),
            pltpu.VMEM((GLEN,), jnp.float32),
            pltpu.VMEM((GLEN,), jnp.float32),
            pltpu.SemaphoreType.DMA,
            pltpu.SemaphoreType.DMA,
            pltpu.SemaphoreType.DMA,
            pltpu.SemaphoreType.DMA,
            pltpu.SemaphoreType.DMA,
        ],
    )(_sc_body)
    return f(xflat, fixflat, ay, ax, dec).reshape(B, C, N)


# speedup vs baseline: 21.0536x; 1.0244x over previous
import functools

import jax
import jax.numpy as jnp
import numpy as np
from jax import lax
from jax.experimental import pallas as pl
from jax.experimental.pallas import tpu as pltpu
from jax.experimental.pallas import tpu_sc as plsc

RES = 256
FOV = 16.0
CMF_A = 0.5
SIGMA = 4.0
B, C, H, W = 32, 3, 512, 512
N = RES * RES
HW = H * W
CHW = C * HW

NUM_CORES = 2
NUM_SUBCORES = 16
NW = NUM_CORES * NUM_SUBCORES
PTS = N // NW
LANES = 16
VSTEPS = PTS // LANES
GLEN = C * PTS
HW16 = HW // NUM_SUBCORES
SKEW = 128
CSTRIDE = HW + SKEW
SBUF = C * CSTRIDE
DEPTH = 3


def _grid_tables():
    r_max = FOV / 2.0
    rho_max = np.log((r_max + CMF_A) / CMF_A)
    lin = np.linspace(-rho_max, rho_max, RES, dtype=np.float32)
    u, v = np.meshgrid(lin, lin, indexing="ij")
    rho = np.sqrt(u ** 2 + v ** 2) + 1e-8
    r = CMF_A * (np.exp(rho) - 1.0)
    r = np.minimum(r, r_max)
    vx = u / rho * r
    vy = v / rho * r
    coords = np.stack([vx.ravel(), vy.ravel()], axis=-1).astype(np.float32) / r_max
    radius = r.ravel().astype(np.float32)
    ay = (coords[:, 0] * np.float32(H - 1)) / np.float32(2.0)
    ax = (coords[:, 1] * np.float32(W - 1)) / np.float32(2.0)
    decay = np.exp(-radius / np.float32(SIGMA)).astype(np.float32)
    return ay.astype(np.float32), ax.astype(np.float32), decay


_AY, _AX, _DECAY = _grid_tables()


def _sc_body(xflat, fixflat, ay_h, ax_h, dec_h, out,
             xs, ay_v, ax_v, dec_v, fix_v,
             i0, i1, g0, g1,
             gs0, gs1, os0, os1, ssa, ssb):
    sid = lax.axis_index("s")
    wid = sid * NUM_CORES + lax.axis_index("c")
    base = wid * PTS

    idxs = (i0, i1)
    gs = (g0, g1)
    gsems = (gs0, gs1)
    osems = (os0, os1)
    ssems = (ssa, ssb)

    pltpu.sync_copy(ay_h.at[pl.ds(base, PTS)], ay_v)
    pltpu.sync_copy(ax_h.at[pl.ds(base, PTS)], ax_v)
    pltpu.sync_copy(dec_h.at[pl.ds(base, PTS)], dec_v)
    pltpu.sync_copy(fixflat, fix_v)

    def fire_stage(b, ssem):
        for c in range(C):
            src = xflat.at[pl.ds(b * CHW + c * HW + sid * HW16, HW16)]
            dst = xs.at[pl.ds((b & 1) * SBUF + c * CSTRIDE + sid * HW16,
                              HW16)]
            pltpu.async_copy(src, dst, ssem)

    def wait_stage(ssem):
        pltpu.make_async_copy(xflat.at[pl.ds(0, C * HW16)],
                              xs.at[pl.ds(0, C * HW16)], ssem).wait()

    def compute_idx(b, idx_v):
        cy = fix_v[pl.ds((2 * b) * LANES, LANES)]
        cx = fix_v[pl.ds((2 * b + 1) * LANES, LANES)]
        qo = (b & 1) * SBUF

        def idx_body(i, _):
            s = i * LANES
            ayv = ay_v[pl.ds(s, LANES)]
            axv = ax_v[pl.ds(s, LANES)]
            uy = jnp.minimum(jnp.maximum(cy + ayv, jnp.float32(0.5)),
                             jnp.float32(H - 1) + jnp.float32(0.5))
            ux = jnp.minimum(jnp.maximum(cx + axv, jnp.float32(0.5)),
                             jnp.float32(W - 1) + jnp.float32(0.5))
            py = uy.astype(jnp.int32)
            px = ux.astype(jnp.int32)
            py = py - jnp.where(py.astype(jnp.float32) == uy, py & 1, 0)
            px = px - jnp.where(px.astype(jnp.float32) == ux, px & 1, 0)
            flat = py * W + px + qo
            idx_v[pl.ds(s, LANES)] = flat
            idx_v[pl.ds(s + PTS, LANES)] = flat + CSTRIDE
            idx_v[pl.ds(s + 2 * PTS, LANES)] = flat + 2 * CSTRIDE
            return 0

        lax.fori_loop(0, VSTEPS, idx_body, 0)

    def fire_gather(idx_v, g_v, gsem):
        pltpu.async_copy(xs.at[idx_v], g_v, gsem)

    def wait_gather(g_v, gsem):
        pltpu.make_async_copy(xflat.at[pl.ds(0, GLEN)], g_v, gsem).wait()

    def decay_mul(g_v):
        def dec_body(i, _):
            s = i * LANES
            d = dec_v[pl.ds(s, LANES)]
            g_v[pl.ds(s, LANES)] = g_v[pl.ds(s, LANES)] * d
            g_v[pl.ds(s + PTS, LANES)] = g_v[pl.ds(s + PTS, LANES)] * d
            g_v[pl.ds(s + 2 * PTS, LANES)] = g_v[pl.ds(s + 2 * PTS, LANES)] * d
            return 0

        lax.fori_loop(0, VSTEPS, dec_body, 0)

    def fire_out(b, g_v, osem):
        obase = b * (C * N) + base
        pltpu.async_copy(g_v.at[pl.ds(0, PTS)], out.at[pl.ds(obase, PTS)], osem)
        pltpu.async_copy(g_v.at[pl.ds(PTS, PTS)],
                         out.at[pl.ds(obase + N, PTS)], osem)
        pltpu.async_copy(g_v.at[pl.ds(2 * PTS, PTS)],
                         out.at[pl.ds(obase + 2 * N, PTS)], osem)

    def wait_out(g_v, osem):
        pltpu.make_async_copy(g_v, out.at[pl.ds(0, GLEN)], osem).wait()

    def body(b, p, do_wait_w=True, do_next=True, do_stage=True, do_idx=True):
        if do_wait_w:
            wait_out(gs[1 - p], osems[1 - p])
        if do_next:
            wait_stage(ssems[1 - p])
            plsc.subcore_barrier()
            fire_gather(idxs[1 - p], gs[1 - p], gsems[1 - p])
        wait_gather(gs[p], gsems[p])
        plsc.subcore_barrier()
        if do_stage:
            fire_stage(b + 2, ssems[p])
        decay_mul(gs[p])
        fire_out(b, gs[p], osems[p])
        if do_idx:
            compute_idx(b + 2, idxs[p])

    fire_stage(0, ssems[0])
    fire_stage(1, ssems[1])
    compute_idx(0, idxs[0])
    compute_idx(1, idxs[1])
    wait_stage(ssems[0])
    plsc.subcore_barrier()
    fire_gather(idxs[0], gs[0], gsems[0])

    body(0, 0, do_wait_w=False)
    body(1, 1)

    def loop(k, _):
        b = 2 * k
        body(b, 0)
        body(b + 1, 1)
        return 0

    lax.fori_loop(1, 15, loop, 0)

    body(30, 0, do_stage=False, do_idx=False)
    body(31, 1, do_next=False, do_stage=False, do_idx=False)
    wait_out(gs[1], osems[1])


@jax.jit
def kernel(x, fix_loc):
    xflat = x.reshape(-1)
    scale = jnp.array([H - 1, W - 1], dtype=jnp.float32)
    fixflat = jnp.repeat((fix_loc * scale + jnp.float32(0.5)).reshape(-1),
                         LANES)
    ay = jnp.asarray(_AY)
    ax = jnp.asarray(_AX)
    dec = jnp.asarray(_DECAY)

    mesh = plsc.VectorSubcoreMesh(core_axis_name="c", subcore_axis_name="s")
    f = functools.partial(
        pl.kernel,
        out_type=jax.ShapeDtypeStruct((B * C * N,), jnp.float32),
        mesh=mesh,
        scratch_types=[
            pltpu.VMEM_SHARED((2 * SBUF,), jnp.float32),
            pltpu.VMEM((PTS,), jnp.float32),
            pltpu.VMEM((PTS,), jnp.float32),
            pltpu.VMEM((PTS,), jnp.float32),
            pltpu.VMEM((2 * B * LANES,), jnp.float32),
            pltpu.VMEM((GLEN,), jnp.int32),
            pltpu.VMEM((GLEN,), jnp.int32),
            pltpu.VMEM((GLEN,), jnp.float32),
            pltpu.VMEM((GLEN,), jnp.float32),
            pltpu.SemaphoreType.DMA,
            pltpu.SemaphoreType.DMA,
            pltpu.SemaphoreType.DMA,
            pltpu.SemaphoreType.DMA,
            pltpu.SemaphoreType.DMA,
            pltpu.SemaphoreType.DMA,
        ],
    )(_sc_body)
    return f(xflat, fixflat, ay, ax, dec).reshape(B, C, N)


# speedup vs baseline: 21.0919x; 1.0018x over previous
import functools

import jax
import jax.numpy as jnp
import numpy as np
from jax import lax
from jax.experimental import pallas as pl
from jax.experimental.pallas import tpu as pltpu
from jax.experimental.pallas import tpu_sc as plsc

RES = 256
FOV = 16.0
CMF_A = 0.5
SIGMA = 4.0
B, C, H, W = 32, 3, 512, 512
N = RES * RES
HW = H * W
CHW = C * HW

NUM_CORES = 2
NUM_SUBCORES = 16
NW = NUM_CORES * NUM_SUBCORES
PTS = N // NW
LANES = 16
VSTEPS = PTS // LANES
GLEN = C * PTS
HW16 = HW // NUM_SUBCORES
SKEW = 128
CSTRIDE = HW + SKEW
SBUF = C * CSTRIDE


def _grid_tables():
    r_max = FOV / 2.0
    rho_max = np.log((r_max + CMF_A) / CMF_A)
    lin = np.linspace(-rho_max, rho_max, RES, dtype=np.float32)
    u, v = np.meshgrid(lin, lin, indexing="ij")
    rho = np.sqrt(u ** 2 + v ** 2) + 1e-8
    r = CMF_A * (np.exp(rho) - 1.0)
    r = np.minimum(r, r_max)
    vx = u / rho * r
    vy = v / rho * r
    coords = np.stack([vx.ravel(), vy.ravel()], axis=-1).astype(np.float32) / r_max
    radius = r.ravel().astype(np.float32)
    ay = (coords[:, 0] * np.float32(H - 1)) / np.float32(2.0)
    ax = (coords[:, 1] * np.float32(W - 1)) / np.float32(2.0)
    decay = np.exp(-radius / np.float32(SIGMA)).astype(np.float32)
    return ay.astype(np.float32), ax.astype(np.float32), decay


_AY, _AX, _DECAY = _grid_tables()


def _sc_body(xflat, fixflat, ay_h, ax_h, dec_h, out,
             xs, ay_v, ax_v, dec_v, fix_v,
             i0, i1, g0, g1,
             gs0, gs1, os0, os1, ssa, ssb):
    sid = lax.axis_index("s")
    wid = sid * NUM_CORES + lax.axis_index("c")
    base = wid * PTS

    idxs = (i0, i1)
    gs = (g0, g1)
    gsems = (gs0, gs1)
    osems = (os0, os1)
    ssems = (ssa, ssb)

    pltpu.sync_copy(ay_h.at[pl.ds(base, PTS)], ay_v)
    pltpu.sync_copy(ax_h.at[pl.ds(base, PTS)], ax_v)
    pltpu.sync_copy(dec_h.at[pl.ds(base, PTS)], dec_v)
    pltpu.sync_copy(fixflat, fix_v)

    def fire_stage(b, ssem):
        for c in range(C):
            src = xflat.at[pl.ds(b * CHW + c * HW + sid * HW16, HW16)]
            dst = xs.at[pl.ds((b & 1) * SBUF + c * CSTRIDE + sid * HW16,
                              HW16)]
            pltpu.async_copy(src, dst, ssem)

    def wait_stage(ssem):
        pltpu.make_async_copy(xflat.at[pl.ds(0, C * HW16)],
                              xs.at[pl.ds(0, C * HW16)], ssem).wait()

    def compute_idx(b, idx_v):
        cy = fix_v[pl.ds((2 * b) * LANES, LANES)]
        cx = fix_v[pl.ds((2 * b + 1) * LANES, LANES)]
        qo = (b & 1) * SBUF

        def idx_body(i, _):
            s = i * LANES
            ayv = ay_v[pl.ds(s, LANES)]
            axv = ax_v[pl.ds(s, LANES)]
            uy = jnp.minimum(jnp.maximum(cy + ayv, jnp.float32(0.5)),
                             jnp.float32(H - 1) + jnp.float32(0.5))
            ux = jnp.minimum(jnp.maximum(cx + axv, jnp.float32(0.5)),
                             jnp.float32(W - 1) + jnp.float32(0.5))
            py = uy.astype(jnp.int32)
            px = ux.astype(jnp.int32)
            py = py - jnp.where(py.astype(jnp.float32) == uy, py & 1, 0)
            px = px - jnp.where(px.astype(jnp.float32) == ux, px & 1, 0)
            flat = py * W + px + qo
            idx_v[pl.ds(s, LANES)] = flat
            return 0

        lax.fori_loop(0, VSTEPS, idx_body, 0)

    def fire_gather(idx_v, g_v, gsem):
        for c in range(C):
            view = xs.at[pl.ds(c * CSTRIDE, 2 * SBUF - 2 * CSTRIDE)]
            pltpu.async_copy(view.at[idx_v], g_v.at[pl.ds(c * PTS, PTS)],
                             gsem)

    def wait_gather(g_v, gsem):
        pltpu.make_async_copy(xflat.at[pl.ds(0, GLEN)], g_v, gsem).wait()

    def decay_mul(g_v):
        def dec_body(i, _):
            s = i * LANES
            d = dec_v[pl.ds(s, LANES)]
            g_v[pl.ds(s, LANES)] = g_v[pl.ds(s, LANES)] * d
            g_v[pl.ds(s + PTS, LANES)] = g_v[pl.ds(s + PTS, LANES)] * d
            g_v[pl.ds(s + 2 * PTS, LANES)] = g_v[pl.ds(s + 2 * PTS, LANES)] * d
            return 0

        lax.fori_loop(0, VSTEPS, dec_body, 0)

    def fire_out(b, g_v, osem):
        obase = b * (C * N) + base
        pltpu.async_copy(g_v.at[pl.ds(0, PTS)], out.at[pl.ds(obase, PTS)], osem)
        pltpu.async_copy(g_v.at[pl.ds(PTS, PTS)],
                         out.at[pl.ds(obase + N, PTS)], osem)
        pltpu.async_copy(g_v.at[pl.ds(2 * PTS, PTS)],
                         out.at[pl.ds(obase + 2 * N, PTS)], osem)

    def wait_out(g_v, osem):
        pltpu.make_async_copy(g_v, out.at[pl.ds(0, GLEN)], osem).wait()

    def body(b, p, do_wait_w=True, do_next=True, do_stage=True, do_idx=True):
        if do_wait_w:
            wait_out(gs[1 - p], osems[1 - p])
        if do_next:
            wait_stage(ssems[1 - p])
            plsc.subcore_barrier()
            fire_gather(idxs[1 - p], gs[1 - p], gsems[1 - p])
        wait_gather(gs[p], gsems[p])
        plsc.subcore_barrier()
        if do_stage:
            fire_stage(b + 2, ssems[p])
        decay_mul(gs[p])
        fire_out(b, gs[p], osems[p])
        if do_idx:
            compute_idx(b + 2, idxs[p])

    fire_stage(0, ssems[0])
    fire_stage(1, ssems[1])
    compute_idx(0, idxs[0])
    compute_idx(1, idxs[1])
    wait_stage(ssems[0])
    plsc.subcore_barrier()
    fire_gather(idxs[0], gs[0], gsems[0])

    body(0, 0, do_wait_w=False)
    body(1, 1)

    def loop(k, _):
        b = 2 * k
        body(b, 0)
        body(b + 1, 1)
        return 0

    lax.fori_loop(1, 15, loop, 0)

    body(30, 0, do_stage=False, do_idx=False)
    body(31, 1, do_next=False, do_stage=False, do_idx=False)
    wait_out(gs[1], osems[1])


@jax.jit
def kernel(x, fix_loc):
    xflat = x.reshape(-1)
    scale = jnp.array([H - 1, W - 1], dtype=jnp.float32)
    fixflat = jnp.repeat((fix_loc * scale + jnp.float32(0.5)).reshape(-1),
                         LANES)
    ay = jnp.asarray(_AY)
    ax = jnp.asarray(_AX)
    dec = jnp.asarray(_DECAY)

    mesh = plsc.VectorSubcoreMesh(core_axis_name="c", subcore_axis_name="s")
    f = functools.partial(
        pl.kernel,
        out_type=jax.ShapeDtypeStruct((B * C * N,), jnp.float32),
        mesh=mesh,
        scratch_types=[
            pltpu.VMEM_SHARED((2 * SBUF,), jnp.float32),
            pltpu.VMEM((PTS,), jnp.float32),
            pltpu.VMEM((PTS,), jnp.float32),
            pltpu.VMEM((PTS,), jnp.float32),
            pltpu.VMEM((2 * B * LANES,), jnp.float32),
            pltpu.VMEM((PTS,), jnp.int32),
            pltpu.VMEM((PTS,), jnp.int32),
            pltpu.VMEM((GLEN,), jnp.float32),
            pltpu.VMEM((GLEN,), jnp.float32),
            pltpu.SemaphoreType.DMA,
            pltpu.SemaphoreType.DMA,
            pltpu.SemaphoreType.DMA,
            pltpu.SemaphoreType.DMA,
            pltpu.SemaphoreType.DMA,
            pltpu.SemaphoreType.DMA,
        ],
    )(_sc_body)
    return f(xflat, fixflat, ay, ax, dec).reshape(B, C, N)
